# Initial kernel scaffold; baseline (speedup 1.0000x reference)
import jax
import jax.numpy as jnp
from jax.experimental import pallas as pl


def kernel(x, edge_index, W, b, gamma, beta):
    raise NotImplementedError("write your pallas kernel here")


# speedup vs baseline: 25.5071x; 25.5071x over previous
import functools

import jax
import jax.numpy as jnp
from jax import lax
from jax.experimental import pallas as pl
from jax.experimental.pallas import tpu as pltpu
from jax.experimental.pallas import tpu_sc as plsc

N = 10000
E = 320000
D = 128
NC = 2
NS = 16
NW = NC * NS
EPW = E // NW
CK = 80
CH = EPW // CK
NPAD = 10240
SA = NPAD // NS

_mesh = plsc.VectorSubcoreMesh(core_axis_name="c", subcore_axis_name="s")


@functools.partial(
    pl.kernel,
    out_type=jax.ShapeDtypeStruct((NC, NPAD), jnp.float32),
    mesh=_mesh,
    scratch_types=[
        pltpu.VMEM((CH, CK), jnp.int32),
        pltpu.VMEM((CK,), jnp.float32),
        pltpu.VMEM_SHARED((NPAD,), jnp.float32),
    ],
)
def _sc_degree(col3d, zpad, dp_out, colv, onesv, deg_sh):
    c = lax.axis_index("c")
    s = lax.axis_index("s")
    w = s * NC + c
    pltpu.sync_copy(zpad.at[pl.ds(s * SA, SA)], deg_sh.at[pl.ds(s * SA, SA)])

    def fill(i, _):
        onesv[pl.ds(i * 16, 16)] = jnp.ones((16,), jnp.float32)
        return 0

    lax.fori_loop(0, CK // 16, fill, 0)
    pltpu.sync_copy(col3d.at[w], colv)
    plsc.subcore_barrier()

    def chunk(j, _):
        pltpu.sync_copy(onesv, deg_sh.at[colv.at[j]], add=True)
        return 0

    lax.fori_loop(0, CH, chunk, 0)
    plsc.subcore_barrier()
    pltpu.sync_copy(deg_sh.at[pl.ds(s * SA, SA)], dp_out.at[c, pl.ds(s * SA, SA)])


@functools.partial(
    pl.kernel,
    out_type=jax.ShapeDtypeStruct((NC, NPAD, D), jnp.float32),
    mesh=_mesh,
    scratch_types=[
        pltpu.VMEM((CH, CK), jnp.int32),
        pltpu.VMEM((CH, CK), jnp.int32),
        pltpu.VMEM((CK, D), jnp.float32),
        pltpu.VMEM_SHARED((NPAD, D), jnp.float32),
    ],
)
def _sc_scatter(row3d, col3d, xs_hbm, zrows, acc_out, rowv, colv, rows_v, acc_sh):
    c = lax.axis_index("c")
    s = lax.axis_index("s")
    w = s * NC + c
    pltpu.sync_copy(zrows, acc_sh.at[pl.ds(s * SA, SA)])
    pltpu.sync_copy(row3d.at[w], rowv)
    pltpu.sync_copy(col3d.at[w], colv)
    plsc.subcore_barrier()

    def chunk(j, _):
        pltpu.sync_copy(xs_hbm.at[rowv.at[j]], rows_v)
        pltpu.sync_copy(rows_v, acc_sh.at[colv.at[j]], add=True)
        return 0

    lax.fori_loop(0, CH, chunk, 0)
    plsc.subcore_barrier()
    pltpu.sync_copy(acc_sh.at[pl.ds(s * SA, SA)], acc_out.at[c, pl.ds(s * SA, SA)])


_BN = 1000
_NB = N // _BN


def _tc_linear_body(x_ref, d0_ref, d1_ref, w_ref, xs_ref, dis_ref):
    deg = d0_ref[...] + d1_ref[...] + 1.0
    dis = lax.rsqrt(deg)
    xw = jnp.dot(x_ref[...], w_ref[...], preferred_element_type=jnp.float32)
    xs_ref[...] = xw * dis
    dis_ref[...] = dis


def _tc_linear(x, d0, d1, w):
    return pl.pallas_call(
        _tc_linear_body,
        grid=(_NB,),
        in_specs=[
            pl.BlockSpec((_BN, D), lambda i: (i, 0)),
            pl.BlockSpec((_BN, 1), lambda i: (i, 0)),
            pl.BlockSpec((_BN, 1), lambda i: (i, 0)),
            pl.BlockSpec((D, D), lambda i: (0, 0)),
        ],
        out_specs=[
            pl.BlockSpec((_BN, D), lambda i: (i, 0)),
            pl.BlockSpec((_BN, 1), lambda i: (i, 0)),
        ],
        out_shape=[
            jax.ShapeDtypeStruct((N, D), jnp.float32),
            jax.ShapeDtypeStruct((N, 1), jnp.float32),
        ],
    )(x, d0, d1, w)


def _tc_combine_body(a0_ref, a1_ref, xs_ref, dis_ref, b_ref, op_ref, st_ref):
    i = pl.program_id(0)
    total = a0_ref[...] + a1_ref[...] + xs_ref[...]
    op = dis_ref[...] * total + b_ref[...]
    op_ref[...] = op

    @pl.when(i == 0)
    def _():
        st_ref[...] = jnp.zeros_like(st_ref)

    st_ref[0:1, :] += jnp.sum(op, axis=0, keepdims=True)
    st_ref[1:2, :] += jnp.sum(op * op, axis=0, keepdims=True)


def _tc_combine(a0, a1, xs, dis, b2):
    return pl.pallas_call(
        _tc_combine_body,
        grid=(_NB,),
        in_specs=[
            pl.BlockSpec((_BN, D), lambda i: (i, 0)),
            pl.BlockSpec((_BN, D), lambda i: (i, 0)),
            pl.BlockSpec((_BN, D), lambda i: (i, 0)),
            pl.BlockSpec((_BN, 1), lambda i: (i, 0)),
            pl.BlockSpec((1, D), lambda i: (0, 0)),
        ],
        out_specs=[
            pl.BlockSpec((_BN, D), lambda i: (i, 0)),
            pl.BlockSpec((8, D), lambda i: (0, 0)),
        ],
        out_shape=[
            jax.ShapeDtypeStruct((N, D), jnp.float32),
            jax.ShapeDtypeStruct((8, D), jnp.float32),
        ],
    )(a0, a1, xs, dis, b2)


def _tc_bn_body(op_ref, st_ref, g_ref, be_ref, o_ref):
    mean = st_ref[0:1, :] * (1.0 / N)
    var = st_ref[1:2, :] * (1.0 / N) - mean * mean
    inv = lax.rsqrt(var + 1e-5)
    o_ref[...] = jnp.maximum((op_ref[...] - mean) * inv * g_ref[...] + be_ref[...], 0.0)


def _tc_bn(op, st, g2, be2):
    return pl.pallas_call(
        _tc_bn_body,
        grid=(_NB,),
        in_specs=[
            pl.BlockSpec((_BN, D), lambda i: (i, 0)),
            pl.BlockSpec((8, D), lambda i: (0, 0)),
            pl.BlockSpec((1, D), lambda i: (0, 0)),
            pl.BlockSpec((1, D), lambda i: (0, 0)),
        ],
        out_specs=pl.BlockSpec((_BN, D), lambda i: (i, 0)),
        out_shape=jax.ShapeDtypeStruct((N, D), jnp.float32),
    )(op, st, g2, be2)


def kernel(x, edge_index, W, b, gamma, beta):
    row3d = edge_index[0].reshape(NW, CH, CK)
    col3d = edge_index[1].reshape(NW, CH, CK)
    zpad = jnp.zeros((NPAD,), jnp.float32)
    zrows = jnp.zeros((SA, D), jnp.float32)

    dp = _sc_degree(col3d, zpad)
    d0 = dp[0, :N].reshape(N, 1)
    d1 = dp[1, :N].reshape(N, 1)

    xs, dis = _tc_linear(x, d0, d1, W)

    acc = _sc_scatter(row3d, col3d, xs, zrows)

    op, st = _tc_combine(acc[0], acc[1], xs, dis, b.reshape(1, D))
    return _tc_bn(op, st, gamma.reshape(1, D), beta.reshape(1, D))


# speedup vs baseline: 36.2192x; 1.4200x over previous
import functools

import jax
import jax.numpy as jnp
from jax import lax
from jax.experimental import pallas as pl
from jax.experimental.pallas import tpu as pltpu
from jax.experimental.pallas import tpu_sc as plsc

N = 10000
E = 320000
D = 128
NC = 2
NS = 16
NW = NC * NS
CK = 128
CH = 80
EPW = CH * CK
EPAD = NW * EPW
NPAD = 10240
SA = NPAD // NS

_mesh = plsc.VectorSubcoreMesh(core_axis_name="c", subcore_axis_name="s")


@functools.partial(
    pl.kernel,
    out_type=jax.ShapeDtypeStruct((NC, NPAD), jnp.float32),
    mesh=_mesh,
    scratch_types=[
        pltpu.VMEM((CH, CK), jnp.int32),
        pltpu.VMEM((CK,), jnp.float32),
        pltpu.VMEM((SA,), jnp.float32),
        pltpu.VMEM_SHARED((NPAD,), jnp.float32),
        pltpu.SemaphoreType.DMA,
    ],
)
def _sc_degree(col3d, dp_out, colv, onesv, zv, deg_sh, sem):
    c = lax.axis_index("c")
    s = lax.axis_index("s")
    w = s * NC + c

    for i in range(CK // 16):
        onesv[pl.ds(i * 16, 16)] = jnp.ones((16,), jnp.float32)
    for i in range(SA // 16):
        zv[pl.ds(i * 16, 16)] = jnp.zeros((16,), jnp.float32)
    pltpu.sync_copy(zv, deg_sh.at[pl.ds(s * SA, SA)])
    pltpu.sync_copy(col3d.at[w], colv)
    plsc.subcore_barrier()

    G = 8

    def group(i, _):
        for t in range(G):
            pltpu.async_copy(onesv, deg_sh.at[colv.at[i * G + t]], sem, add=True)
        for t in range(G):
            pltpu.make_async_copy(onesv, deg_sh.at[colv.at[i * G + t]], sem).wait()
        return 0

    lax.fori_loop(0, CH // G, group, 0)
    plsc.subcore_barrier()
    pltpu.sync_copy(deg_sh.at[pl.ds(s * SA, SA)], dp_out.at[c, pl.ds(s * SA, SA)])


@functools.partial(
    pl.kernel,
    out_type=jax.ShapeDtypeStruct((NC, NPAD, D), jnp.float32),
    mesh=_mesh,
    scratch_types=[
        pltpu.VMEM((CH // 2, CK), jnp.int32),
        pltpu.VMEM((CH // 2, CK), jnp.int32),
        pltpu.VMEM((CK, D), jnp.float32),
        pltpu.VMEM((CK, D), jnp.float32),
        pltpu.VMEM_SHARED((NPAD, D), jnp.float32),
        pltpu.SemaphoreType.DMA,
        pltpu.SemaphoreType.DMA,
    ],
)
def _sc_scatter(row3d, col3d, xs_hbm, zc, acc_out, rowv, colv, r0, r1, acc_sh, g0, g1):
    c = lax.axis_index("c")
    s = lax.axis_index("s")
    w = s * NC + c
    HC = CH // 2
    pltpu.sync_copy(zc, r0)
    for t in range(SA // CK):
        pltpu.sync_copy(r0, acc_sh.at[pl.ds(s * SA + t * CK, CK)])
    plsc.subcore_barrier()

    def body(i, _):
        a = 2 * i
        b = a + 1
        pltpu.make_async_copy(xs_hbm.at[rowv.at[a]], r0, g0).wait()
        pltpu.async_copy(xs_hbm.at[rowv.at[b]], r1, g1)
        pltpu.sync_copy(r0, acc_sh.at[colv.at[a]], add=True)
        pltpu.make_async_copy(xs_hbm.at[rowv.at[b]], r1, g1).wait()
        nxt = jnp.minimum(a + 2, HC - 1)
        pltpu.async_copy(xs_hbm.at[rowv.at[nxt]], r0, g0)
        pltpu.sync_copy(r1, acc_sh.at[colv.at[b]], add=True)
        return 0

    for p in range(2):
        pltpu.sync_copy(row3d.at[w, pl.ds(p * HC, HC)], rowv)
        pltpu.sync_copy(col3d.at[w, pl.ds(p * HC, HC)], colv)
        pltpu.async_copy(xs_hbm.at[rowv.at[0]], r0, g0)
        lax.fori_loop(0, HC // 2, body, 0)
        pltpu.make_async_copy(xs_hbm.at[rowv.at[HC - 1]], r0, g0).wait()
    plsc.subcore_barrier()
    pltpu.sync_copy(acc_sh.at[pl.ds(s * SA, SA)], acc_out.at[c, pl.ds(s * SA, SA)])


_BN = 1000
_NB = N // _BN


def _tc_xw_body(x_ref, w_ref, xw_ref):
    xw_ref[...] = jnp.dot(x_ref[...], w_ref[...], preferred_element_type=jnp.float32)


def _tc_xw(x, w):
    return pl.pallas_call(
        _tc_xw_body,
        grid=(_NB,),
        in_specs=[
            pl.BlockSpec((_BN, D), lambda i: (i, 0)),
            pl.BlockSpec((D, D), lambda i: (0, 0)),
        ],
        out_specs=pl.BlockSpec((_BN, D), lambda i: (i, 0)),
        out_shape=jax.ShapeDtypeStruct((N, D), jnp.float32),
    )(x, w)


def _tc_scale_body(xw_ref, d0_ref, d1_ref, xs_ref, dis_ref):
    deg = d0_ref[...] + d1_ref[...] + 1.0
    dis = lax.rsqrt(deg)
    xs_ref[...] = xw_ref[...] * dis
    dis_ref[...] = dis


def _tc_scale(xw, d0, d1):
    return pl.pallas_call(
        _tc_scale_body,
        grid=(_NB,),
        in_specs=[
            pl.BlockSpec((_BN, D), lambda i: (i, 0)),
            pl.BlockSpec((_BN, 1), lambda i: (i, 0)),
            pl.BlockSpec((_BN, 1), lambda i: (i, 0)),
        ],
        out_specs=[
            pl.BlockSpec((_BN, D), lambda i: (i, 0)),
            pl.BlockSpec((_BN, 1), lambda i: (i, 0)),
        ],
        out_shape=[
            jax.ShapeDtypeStruct((N, D), jnp.float32),
            jax.ShapeDtypeStruct((N, 1), jnp.float32),
        ],
    )(xw, d0, d1)


def _tc_combine_body(a0_ref, a1_ref, xs_ref, dis_ref, b_ref, op_ref, st_ref):
    i = pl.program_id(0)
    total = a0_ref[0] + a1_ref[0] + xs_ref[...]
    op = dis_ref[...] * total + b_ref[...]
    op_ref[...] = op

    @pl.when(i == 0)
    def _():
        st_ref[...] = jnp.zeros_like(st_ref)

    st_ref[0:1, :] += jnp.sum(op, axis=0, keepdims=True)
    st_ref[1:2, :] += jnp.sum(op * op, axis=0, keepdims=True)


def _tc_combine(acc, xs, dis, b2):
    return pl.pallas_call(
        _tc_combine_body,
        grid=(_NB,),
        in_specs=[
            pl.BlockSpec((1, _BN, D), lambda i: (0, i, 0)),
            pl.BlockSpec((1, _BN, D), lambda i: (1, i, 0)),
            pl.BlockSpec((_BN, D), lambda i: (i, 0)),
            pl.BlockSpec((_BN, 1), lambda i: (i, 0)),
            pl.BlockSpec((1, D), lambda i: (0, 0)),
        ],
        out_specs=[
            pl.BlockSpec((_BN, D), lambda i: (i, 0)),
            pl.BlockSpec((8, D), lambda i: (0, 0)),
        ],
        out_shape=[
            jax.ShapeDtypeStruct((N, D), jnp.float32),
            jax.ShapeDtypeStruct((8, D), jnp.float32),
        ],
    )(acc, acc, xs, dis, b2)


def _tc_bn_body(op_ref, st_ref, g_ref, be_ref, o_ref):
    mean = st_ref[0:1, :] * (1.0 / N)
    var = st_ref[1:2, :] * (1.0 / N) - mean * mean
    inv = lax.rsqrt(var + 1e-5)
    o_ref[...] = jnp.maximum((op_ref[...] - mean) * inv * g_ref[...] + be_ref[...], 0.0)


def _tc_bn(op, st, g2, be2):
    return pl.pallas_call(
        _tc_bn_body,
        grid=(_NB,),
        in_specs=[
            pl.BlockSpec((_BN, D), lambda i: (i, 0)),
            pl.BlockSpec((8, D), lambda i: (0, 0)),
            pl.BlockSpec((1, D), lambda i: (0, 0)),
            pl.BlockSpec((1, D), lambda i: (0, 0)),
        ],
        out_specs=pl.BlockSpec((_BN, D), lambda i: (i, 0)),
        out_shape=jax.ShapeDtypeStruct((N, D), jnp.float32),
    )(op, st, g2, be2)


def kernel(x, edge_index, W, b, gamma, beta):
    npe = EPAD - E
    prow = jnp.arange(npe, dtype=jnp.int32)
    pcol = N + prow % jnp.int32(NPAD - N)
    row3d = jnp.concatenate([edge_index[0], prow]).reshape(NW, CH, CK)
    col3d = jnp.concatenate([edge_index[1], pcol]).reshape(NW, CH, CK)
    zc = jnp.zeros((CK, D), jnp.float32)

    dp = _sc_degree(col3d)
    d0 = dp[0, :N].reshape(N, 1)
    d1 = dp[1, :N].reshape(N, 1)

    xw = _tc_xw(x, W)
    xs, dis = _tc_scale(xw, d0, d1)

    acc = _sc_scatter(row3d, col3d, xs, zc)

    op, st = _tc_combine(acc, xs, dis, b.reshape(1, D))
    return _tc_bn(op, st, gamma.reshape(1, D), beta.reshape(1, D))
